# baseline (speedup 1.0000x reference)
import functools

import jax
import jax.numpy as jnp
from jax import lax
from jax.experimental import pallas as pl
from jax.experimental.pallas import tpu as pltpu
from jax.experimental.pallas import tpu_sc as plsc

N = 10000
E = 320000
IN_DIM = 160
HALF = 80
FEAT = 128

EDGES_PER_TILE = E // 16
CHUNK = 80
NCHUNKS = EDGES_PER_TILE // CHUNK
ROWS_PER_SUB = N // 16
CNT_W = 16
NBUF = 4
HCH = NCHUNKS // 2


def _sc_aggregate(x0, x1, src_e, dst_e):
  mesh = plsc.VectorSubcoreMesh(core_axis_name="c", subcore_axis_name="s")
  zrow = jnp.zeros((ROWS_PER_SUB, HALF), jnp.float32)
  zcnt = jnp.zeros((ROWS_PER_SUB, CNT_W), jnp.float32)
  ones = jnp.ones((CHUNK, CNT_W), jnp.float32)

  @functools.partial(
      pl.kernel,
      out_type=[
          jax.ShapeDtypeStruct((2, N, HALF), jnp.float32),
          jax.ShapeDtypeStruct((2, N, CNT_W), jnp.float32),
      ],
      mesh=mesh,
      compiler_params=pltpu.CompilerParams(use_tc_tiling_on_sc=False),
      scratch_types=[
          pltpu.VMEM((NCHUNKS, CHUNK), jnp.int32),
          pltpu.VMEM((NCHUNKS, CHUNK), jnp.int32),
          [pltpu.VMEM((CHUNK, HALF), jnp.float32) for _ in range(NBUF)],
          pltpu.VMEM((CHUNK, CNT_W), jnp.float32),
          pltpu.VMEM_SHARED((N, HALF), jnp.float32),
          pltpu.VMEM_SHARED((N, CNT_W), jnp.float32),
          [pltpu.SemaphoreType.DMA for _ in range(NBUF)],
          [pltpu.SemaphoreType.DMA for _ in range(NBUF)],
          [pltpu.SemaphoreType.DMA for _ in range(NBUF)],
      ],
  )
  def agg_kernel(x0_hbm, x1_hbm, src_hbm, dst_hbm, zrow_hbm, zcnt_hbm,
                 ones_hbm, agg_out, cnt_out,
                 src_v, dst_v, rows, ones_v, agg_s, cnt_s,
                 gsem, ssem, csem):
    c = lax.axis_index("c")
    s = lax.axis_index("s")
    base = s * ROWS_PER_SUB

    pltpu.sync_copy(src_hbm.at[s], src_v)
    pltpu.sync_copy(dst_hbm.at[s], dst_v)
    pltpu.sync_copy(ones_hbm, ones_v)

    pltpu.sync_copy(zrow_hbm, agg_s.at[pl.ds(base, ROWS_PER_SUB)])
    pltpu.sync_copy(zcnt_hbm, cnt_s.at[pl.ds(base, ROWS_PER_SUB)])

    def sidx(j):
      return src_v.at[j]

    def didx(j):
      return dst_v.at[j]

    def start_gather(j, b):
      @pl.when(c == 0)
      def _():
        pltpu.make_async_copy(x0_hbm.at[sidx(j)], rows[b], gsem[b]).start()
      @pl.when(c == 1)
      def _():
        pltpu.make_async_copy(x1_hbm.at[sidx(j)], rows[b], gsem[b]).start()

    def wait_gather(j, b):
      pltpu.make_async_copy(x0_hbm.at[sidx(j)], rows[b], gsem[b]).wait()

    def cnt_cond(j):
      return (j >= c * HCH) & (j < (c + 1) * HCH)

    def scatter(j, buf):
      pltpu.sync_copy(buf, agg_s.at[didx(j)], add=True)
      @pl.when(cnt_cond(j))
      def _():
        pltpu.sync_copy(ones_v, cnt_s.at[didx(j)], add=True)

    start_gather(0, 0)
    plsc.subcore_barrier()

    def body(g, carry):
      j0 = 2 * g
      wait_gather(j0, 0)
      start_gather(j0 + 1, 1)
      scatter(j0, rows[0])
      wait_gather(j0 + 1, 1)
      start_gather(j0 + 2, 0)
      scatter(j0 + 1, rows[1])
      return carry

    lax.fori_loop(0, NCHUNKS // 2 - 1, body, 0)
    wait_gather(NCHUNKS - 2, 0)
    start_gather(NCHUNKS - 1, 1)
    scatter(NCHUNKS - 2, rows[0])
    wait_gather(NCHUNKS - 1, 1)
    scatter(NCHUNKS - 1, rows[1])

    plsc.subcore_barrier()
    pltpu.sync_copy(agg_s.at[pl.ds(base, ROWS_PER_SUB)],
                    agg_out.at[c, pl.ds(base, ROWS_PER_SUB)])
    pltpu.sync_copy(cnt_s.at[pl.ds(base, ROWS_PER_SUB)],
                    cnt_out.at[c, pl.ds(base, ROWS_PER_SUB)])

  return agg_kernel(x0, x1, src_e, dst_e, zrow, zcnt, ones)


def _tc_body(agg_ref, cnt_ref, nf_ref, nz_ref, wl_ref, wr_ref, b_ref, g_ref,
             be_ref, w1_ref, b1_ref, w2_ref, b2_ref, w3_ref, b3_ref, w4_ref,
             b4_ref, out_ref):
  inv = 1.0 / jnp.maximum(cnt_ref[0, :, 0:1] + cnt_ref[1, :, 0:1], 1.0)
  h = (jnp.dot(agg_ref[0] * inv, wl_ref[0:HALF],
               preferred_element_type=jnp.float32)
       + jnp.dot(agg_ref[1] * inv, wl_ref[HALF:IN_DIM],
                 preferred_element_type=jnp.float32)
       + jnp.dot(nf_ref[...], wr_ref[0:FEAT],
                 preferred_element_type=jnp.float32)
       + jnp.dot(nz_ref[...], wr_ref[FEAT:IN_DIM],
                 preferred_element_type=jnp.float32)
       + b_ref[...])
  mu = jnp.mean(h, axis=0, keepdims=True)
  var = jnp.mean(h * h, axis=0, keepdims=True) - mu * mu
  h = (h - mu) * (g_ref[...] * lax.rsqrt(var + 1e-5)) + be_ref[...]
  h = jnp.where(h > 0, h, 0.2 * h)
  h = jnp.dot(h, w1_ref[...], preferred_element_type=jnp.float32) + b1_ref[...]
  h = jnp.where(h > 0, h, 0.2 * h)
  h = jnp.dot(h, w2_ref[...], preferred_element_type=jnp.float32) + b2_ref[...]
  h = jnp.where(h > 0, h, 0.2 * h)
  h = jnp.dot(h, w3_ref[...], preferred_element_type=jnp.float32) + b3_ref[...]
  h = jnp.where(h > 0, h, 0.2 * h)
  h = jnp.dot(h, w4_ref[...], preferred_element_type=jnp.float32) + b4_ref[...]
  out_ref[...] = jnp.where(h > 0, h, 0.2 * h)


def kernel(normal_features, noise, edge_index, batch, W_l, W_r, b_sage,
           gamma, beta, fc1_W, fc1_b, fc2_W, fc2_b, fc3_W, fc3_b,
           fc4_W, fc4_b):
  del batch
  x0 = normal_features[:, :HALF]
  x1 = jnp.concatenate([normal_features[:, HALF:], noise], axis=1)
  agg, cnt = _sc_aggregate(x0, x1,
                           edge_index[0].reshape(16, NCHUNKS, CHUNK),
                           edge_index[1].reshape(16, NCHUNKS, CHUNK))
  return pl.pallas_call(
      _tc_body,
      out_shape=jax.ShapeDtypeStruct((N, FEAT), jnp.float32),
  )(agg, cnt, normal_features, noise, W_l, W_r, b_sage, gamma, beta,
    fc1_W, fc1_b, fc2_W, fc2_b, fc3_W, fc3_b, fc4_W, fc4_b)


# speedup vs baseline: 1.5559x; 1.5559x over previous
import functools

import jax
import jax.numpy as jnp
from jax import lax
from jax.experimental import pallas as pl
from jax.experimental.pallas import tpu as pltpu
from jax.experimental.pallas import tpu_sc as plsc

N = 10000
E = 320000
IN_DIM = 160
HALF = 80
FEAT = 128

EDGES_PER_TILE = E // 16
CHUNK = 80
NCHUNKS = EDGES_PER_TILE // CHUNK
ROWS_PER_SUB = N // 16
CNT_W = 16
NBUF = 4
HCH = NCHUNKS // 2


def _sc_aggregate(x0, x1, src_e, dst_e):
  mesh = plsc.VectorSubcoreMesh(core_axis_name="c", subcore_axis_name="s")
  zrow = jnp.zeros((ROWS_PER_SUB, HALF), jnp.float32)
  zcnt = jnp.zeros((ROWS_PER_SUB, CNT_W), jnp.float32)
  ones = jnp.ones((CHUNK, CNT_W), jnp.float32)

  @functools.partial(
      pl.kernel,
      out_type=[
          jax.ShapeDtypeStruct((2, N, HALF), jnp.float32),
          jax.ShapeDtypeStruct((2, N, CNT_W), jnp.float32),
      ],
      mesh=mesh,
      compiler_params=pltpu.CompilerParams(use_tc_tiling_on_sc=False),
      scratch_types=[
          pltpu.VMEM((NCHUNKS, CHUNK), jnp.int32),
          pltpu.VMEM((NCHUNKS, CHUNK), jnp.int32),
          [pltpu.VMEM((CHUNK, HALF), jnp.float32) for _ in range(NBUF)],
          pltpu.VMEM((CHUNK, CNT_W), jnp.float32),
          pltpu.VMEM_SHARED((N, HALF), jnp.float32),
          pltpu.VMEM_SHARED((N, CNT_W), jnp.float32),
          [pltpu.SemaphoreType.DMA for _ in range(NBUF)],
          [pltpu.SemaphoreType.DMA for _ in range(NBUF)],
          [pltpu.SemaphoreType.DMA for _ in range(NBUF)],
      ],
  )
  def agg_kernel(x0_hbm, x1_hbm, src_hbm, dst_hbm, zrow_hbm, zcnt_hbm,
                 ones_hbm, agg_out, cnt_out,
                 src_v, dst_v, rows, ones_v, agg_s, cnt_s,
                 gsem, ssem, csem):
    c = lax.axis_index("c")
    s = lax.axis_index("s")
    base = s * ROWS_PER_SUB

    pltpu.sync_copy(src_hbm.at[s], src_v)
    pltpu.sync_copy(dst_hbm.at[s], dst_v)
    pltpu.sync_copy(ones_hbm, ones_v)

    pltpu.sync_copy(zrow_hbm, agg_s.at[pl.ds(base, ROWS_PER_SUB)])
    pltpu.sync_copy(zcnt_hbm, cnt_s.at[pl.ds(base, ROWS_PER_SUB)])

    def sidx(j):
      return src_v.at[j]

    def didx(j):
      return dst_v.at[j]

    def start_gather(j, b):
      @pl.when(c == 0)
      def _():
        pltpu.make_async_copy(x0_hbm.at[sidx(j)], rows[b], gsem[b]).start()
      @pl.when(c == 1)
      def _():
        pltpu.make_async_copy(x1_hbm.at[sidx(j)], rows[b], gsem[b]).start()

    def wait_gather(j, b):
      pltpu.make_async_copy(x0_hbm.at[sidx(j)], rows[b], gsem[b]).wait()

    def cnt_cond(j):
      return (j >= c * HCH) & (j < (c + 1) * HCH)

    def scatter(j, buf):
      pltpu.sync_copy(buf, agg_s.at[didx(j)], add=True)
      @pl.when(cnt_cond(j))
      def _():
        pltpu.sync_copy(ones_v, cnt_s.at[didx(j)], add=True)

    start_gather(0, 0)
    start_gather(1, 1)
    start_gather(2, 2)
    plsc.subcore_barrier()

    def body(g, carry):
      j0 = NBUF * g
      for b in range(NBUF):
        j = j0 + b
        @pl.when(j < NCHUNKS)
        def _():
          wait_gather(j, b)
          @pl.when(j + 3 < NCHUNKS)
          def _():
            start_gather(j + 3, (b + 3) % NBUF)
          scatter(j, rows[b])
      return carry

    lax.fori_loop(0, (NCHUNKS + NBUF - 1) // NBUF, body, 0)

    plsc.subcore_barrier()
    pltpu.sync_copy(agg_s.at[pl.ds(base, ROWS_PER_SUB)],
                    agg_out.at[c, pl.ds(base, ROWS_PER_SUB)])
    pltpu.sync_copy(cnt_s.at[pl.ds(base, ROWS_PER_SUB)],
                    cnt_out.at[c, pl.ds(base, ROWS_PER_SUB)])

  return agg_kernel(x0, x1, src_e, dst_e, zrow, zcnt, ones)


def _tc_body(agg_ref, cnt_ref, nf_ref, nz_ref, wl_ref, wr_ref, b_ref, g_ref,
             be_ref, w1_ref, b1_ref, w2_ref, b2_ref, w3_ref, b3_ref, w4_ref,
             b4_ref, out_ref):
  inv = 1.0 / jnp.maximum(cnt_ref[0, :, 0:1] + cnt_ref[1, :, 0:1], 1.0)
  h = (jnp.dot(agg_ref[0] * inv, wl_ref[0:HALF],
               preferred_element_type=jnp.float32)
       + jnp.dot(agg_ref[1] * inv, wl_ref[HALF:IN_DIM],
                 preferred_element_type=jnp.float32)
       + jnp.dot(nf_ref[...], wr_ref[0:FEAT],
                 preferred_element_type=jnp.float32)
       + jnp.dot(nz_ref[...], wr_ref[FEAT:IN_DIM],
                 preferred_element_type=jnp.float32)
       + b_ref[...])
  mu = jnp.mean(h, axis=0, keepdims=True)
  var = jnp.mean(h * h, axis=0, keepdims=True) - mu * mu
  h = (h - mu) * (g_ref[...] * lax.rsqrt(var + 1e-5)) + be_ref[...]
  h = jnp.where(h > 0, h, 0.2 * h)
  h = jnp.dot(h, w1_ref[...], preferred_element_type=jnp.float32) + b1_ref[...]
  h = jnp.where(h > 0, h, 0.2 * h)
  h = jnp.dot(h, w2_ref[...], preferred_element_type=jnp.float32) + b2_ref[...]
  h = jnp.where(h > 0, h, 0.2 * h)
  h = jnp.dot(h, w3_ref[...], preferred_element_type=jnp.float32) + b3_ref[...]
  h = jnp.where(h > 0, h, 0.2 * h)
  h = jnp.dot(h, w4_ref[...], preferred_element_type=jnp.float32) + b4_ref[...]
  out_ref[...] = jnp.where(h > 0, h, 0.2 * h)


def kernel(normal_features, noise, edge_index, batch, W_l, W_r, b_sage,
           gamma, beta, fc1_W, fc1_b, fc2_W, fc2_b, fc3_W, fc3_b,
           fc4_W, fc4_b):
  del batch
  x0 = normal_features[:, :HALF]
  x1 = jnp.concatenate([normal_features[:, HALF:], noise], axis=1)
  agg, cnt = _sc_aggregate(x0, x1,
                           edge_index[0].reshape(16, NCHUNKS, CHUNK),
                           edge_index[1].reshape(16, NCHUNKS, CHUNK))
  return pl.pallas_call(
      _tc_body,
      out_shape=jax.ShapeDtypeStruct((N, FEAT), jnp.float32),
  )(agg, cnt, normal_features, noise, W_l, W_r, b_sage, gamma, beta,
    fc1_W, fc1_b, fc2_W, fc2_b, fc3_W, fc3_b, fc4_W, fc4_b)


# speedup vs baseline: 1.7190x; 1.1048x over previous
import functools

import jax
import jax.numpy as jnp
from jax import lax
from jax.experimental import pallas as pl
from jax.experimental.pallas import tpu as pltpu
from jax.experimental.pallas import tpu_sc as plsc

N = 10000
E = 320000
IN_DIM = 160
HALF = 80
FEAT = 128

EDGES_PER_TILE = E // 16
CHUNK = 80
NCHUNKS = EDGES_PER_TILE // CHUNK
ROWS_PER_SUB = N // 16
CNT_W = 16
NBUF = 4
HCH = NCHUNKS // 2


def _sc_aggregate(x0, x1, src_e, dst_e):
  mesh = plsc.VectorSubcoreMesh(core_axis_name="c", subcore_axis_name="s")
  zrow = jnp.zeros((ROWS_PER_SUB, HALF), jnp.float32)
  zcnt = jnp.zeros((ROWS_PER_SUB, CNT_W), jnp.float32)
  ones = jnp.ones((CHUNK, CNT_W), jnp.float32)

  @functools.partial(
      pl.kernel,
      out_type=jax.ShapeDtypeStruct((2, N, FEAT), jnp.float32),
      mesh=mesh,
      compiler_params=pltpu.CompilerParams(use_tc_tiling_on_sc=False),
      scratch_types=[
          pltpu.VMEM((NCHUNKS, CHUNK), jnp.int32),
          pltpu.VMEM((NCHUNKS, CHUNK), jnp.int32),
          [pltpu.VMEM((CHUNK, HALF), jnp.float32) for _ in range(NBUF)],
          pltpu.VMEM((CHUNK, CNT_W), jnp.float32),
          pltpu.VMEM_SHARED((N, HALF), jnp.float32),
          pltpu.VMEM_SHARED((N, CNT_W), jnp.float32),
          [pltpu.SemaphoreType.DMA for _ in range(NBUF)],
          [pltpu.SemaphoreType.DMA for _ in range(NBUF)],
          [pltpu.SemaphoreType.DMA for _ in range(NBUF)],
      ],
  )
  def agg_kernel(x0_hbm, x1_hbm, src_hbm, dst_hbm, zrow_hbm, zcnt_hbm,
                 ones_hbm, out,
                 src_v, dst_v, rows, ones_v, agg_s, cnt_s,
                 gsem, ssem, csem):
    c = lax.axis_index("c")
    s = lax.axis_index("s")
    base = s * ROWS_PER_SUB

    pltpu.sync_copy(src_hbm.at[s], src_v)
    pltpu.sync_copy(dst_hbm.at[s], dst_v)
    pltpu.sync_copy(ones_hbm, ones_v)

    pltpu.sync_copy(zrow_hbm, agg_s.at[pl.ds(base, ROWS_PER_SUB)])
    pltpu.sync_copy(zcnt_hbm, cnt_s.at[pl.ds(base, ROWS_PER_SUB)])

    def sidx(j):
      return src_v.at[j]

    def didx(j):
      return dst_v.at[j]

    def start_gather(j, b):
      @pl.when(c == 0)
      def _():
        pltpu.make_async_copy(x0_hbm.at[sidx(j)], rows[b], gsem[b]).start()
      @pl.when(c == 1)
      def _():
        pltpu.make_async_copy(x1_hbm.at[sidx(j)], rows[b], gsem[b]).start()

    def wait_gather(j, b):
      pltpu.make_async_copy(x0_hbm.at[sidx(j)], rows[b], gsem[b]).wait()

    def cnt_cond(j):
      return (j >= c * HCH) & (j < (c + 1) * HCH)

    def scatter(j, buf):
      pltpu.sync_copy(buf, agg_s.at[didx(j)], add=True)
      @pl.when(cnt_cond(j))
      def _():
        pltpu.sync_copy(ones_v, cnt_s.at[didx(j)], add=True)

    start_gather(0, 0)
    start_gather(1, 1)
    start_gather(2, 2)
    plsc.subcore_barrier()

    def body(g, carry):
      j0 = NBUF * g
      for b in range(NBUF):
        j = j0 + b
        @pl.when(j < NCHUNKS)
        def _():
          wait_gather(j, b)
          @pl.when(j + 3 < NCHUNKS)
          def _():
            start_gather(j + 3, (b + 3) % NBUF)
          scatter(j, rows[b])
      return carry

    lax.fori_loop(0, (NCHUNKS + NBUF - 1) // NBUF, body, 0)

    plsc.subcore_barrier()
    pltpu.sync_copy(agg_s.at[pl.ds(base, ROWS_PER_SUB)],
                    out.at[c, pl.ds(base, ROWS_PER_SUB), pl.ds(0, HALF)])
    pltpu.sync_copy(cnt_s.at[pl.ds(base, ROWS_PER_SUB)],
                    out.at[c, pl.ds(base, ROWS_PER_SUB), pl.ds(HALF, CNT_W)])

  return agg_kernel(x0, x1, src_e, dst_e, zrow, zcnt, ones)


def _tc_body(sc_ref, nf_ref, nz_ref, wl_ref, wr_ref, b_ref, g_ref,
             be_ref, w1_ref, b1_ref, w2_ref, b2_ref, w3_ref, b3_ref, w4_ref,
             b4_ref, out_ref):
  p0 = sc_ref[0]
  p1 = sc_ref[1]
  inv = 1.0 / jnp.maximum(p0[:, HALF:HALF + 1] + p1[:, HALF:HALF + 1], 1.0)
  h = (jnp.dot(p0[:, 0:HALF] * inv, wl_ref[0:HALF],
               preferred_element_type=jnp.float32)
       + jnp.dot(p1[:, 0:HALF] * inv, wl_ref[HALF:IN_DIM],
                 preferred_element_type=jnp.float32)
       + jnp.dot(nf_ref[...], wr_ref[0:FEAT],
                 preferred_element_type=jnp.float32)
       + jnp.dot(nz_ref[...], wr_ref[FEAT:IN_DIM],
                 preferred_element_type=jnp.float32)
       + b_ref[...])
  mu = jnp.mean(h, axis=0, keepdims=True)
  var = jnp.mean(h * h, axis=0, keepdims=True) - mu * mu
  h = (h - mu) * (g_ref[...] * lax.rsqrt(var + 1e-5)) + be_ref[...]
  h = jnp.where(h > 0, h, 0.2 * h)
  h = jnp.dot(h, w1_ref[...], preferred_element_type=jnp.float32) + b1_ref[...]
  h = jnp.where(h > 0, h, 0.2 * h)
  h = jnp.dot(h, w2_ref[...], preferred_element_type=jnp.float32) + b2_ref[...]
  h = jnp.where(h > 0, h, 0.2 * h)
  h = jnp.dot(h, w3_ref[...], preferred_element_type=jnp.float32) + b3_ref[...]
  h = jnp.where(h > 0, h, 0.2 * h)
  h = jnp.dot(h, w4_ref[...], preferred_element_type=jnp.float32) + b4_ref[...]
  out_ref[...] = jnp.where(h > 0, h, 0.2 * h)


def kernel(normal_features, noise, edge_index, batch, W_l, W_r, b_sage,
           gamma, beta, fc1_W, fc1_b, fc2_W, fc2_b, fc3_W, fc3_b,
           fc4_W, fc4_b):
  del batch
  x0 = normal_features[:, :HALF]
  x1 = jnp.concatenate([normal_features[:, HALF:], noise], axis=1)
  sc_out = _sc_aggregate(x0, x1,
                         edge_index[0].reshape(16, NCHUNKS, CHUNK),
                         edge_index[1].reshape(16, NCHUNKS, CHUNK))
  return pl.pallas_call(
      _tc_body,
      out_shape=jax.ShapeDtypeStruct((N, FEAT), jnp.float32),
  )(sc_out, normal_features, noise, W_l, W_r, b_sage, gamma, beta,
    fc1_W, fc1_b, fc2_W, fc2_b, fc3_W, fc3_b, fc4_W, fc4_b)
